# baseline (speedup 1.0000x reference)
import jax
import jax.numpy as jnp
from jax import lax
from jax.experimental import pallas as pl
from jax.experimental.pallas import tpu as pltpu
from jax.experimental.pallas import tpu_sc as plsc

K = 512
B, N = 128, 32768
NC, NS, L = 2, 16, 16
NW = NC * NS
RPW = B // NW
NV = N // L
PL = N // L
MASK7F = 0x7FFFFFFF
MININT = -2147483648
FF = 0xFF


def _key(b):
    m = jnp.right_shift(b, 31)
    return b ^ (~m & MASK7F)


def _locate(gt, hist_ref, r, L=16):
    cst = plsc.cumsum(gt)
    mlt = cst < r
    gs = plsc.all_reduce_population_count(mlt)[0]
    run = jnp.max(jnp.where(mlt, cst, 0))
    v = hist_ref[pl.ds(gs * L, L)]
    cs = plsc.cumsum(v) + run
    m2 = cs < r
    bw = plsc.all_reduce_population_count(m2)[0]
    habove = jnp.max(jnp.where(m2, cs, run))
    return gs * L + bw, habove


def _sc_body(x_hbm, out_hbm, rowa_v, rowb_v, cand_v, lh_v, merged_v, hist_v,
             gtot_v, ghist_v, pack_v, sema, semb):
    wid = lax.axis_index("s") * NC + lax.axis_index("c")
    lanes = lax.iota(jnp.int32, L)
    ones = jnp.ones((L,), jnp.int32)
    zvec = jnp.zeros((L,), jnp.int32)

    rows = [rowa_v, rowb_v]
    sems = [sema, semb]
    copies = [None, None]
    copies[0] = pltpu.async_copy(x_hbm.at[wid * RPW], rowa_v, sema)

    def zero_hist():
        for g in range(256 // L):
            hist_v[pl.ds(g * L, L)] = zvec

    pack = jnp.zeros((L,), jnp.int32)
    for rr in range(RPW):
        row_v = rows[rr % 2]
        if rr + 1 < RPW:
            copies[(rr + 1) % 2] = pltpu.async_copy(
                x_hbm.at[wid * RPW + rr + 1], rows[(rr + 1) % 2],
                sems[(rr + 1) % 2])
        copies[rr % 2].wait()

        @plsc.parallel_loop(0, 256 * L // L, unroll=4)
        def _(g):
            lh_v[pl.ds(g * L, L)] = zvec

        lane_base = lanes * 256

        lb127 = lane_base + 127

        @plsc.parallel_loop(0, NV, unroll=8)
        def _(i):
            b = plsc.bitcast(row_v[pl.ds(i * L, L)], jnp.int32)
            hb = lax.shift_right_logical(b, 24)
            idx = jnp.where(b < 0, lane_base + hb, lb127 - hb)
            plsc.addupdate_scatter(lh_v, [idx], ones)

        r = jnp.int32(K)
        lane0 = lanes == 0

        @plsc.parallel_loop(0, 256 // L, unroll=2)
        def _(g):
            v = lh_v[pl.ds(g * L, L)]
            for l in range(1, L):
                v = v + lh_v[pl.ds(l * 256 + g * L, L)]
            merged_v[pl.ds(g * L, L)] = v
            tot = jnp.sum(v)
            plsc.store_scatter(gtot_v, [zvec + g], zvec + tot, mask=lane0)

        bsel, habove = _locate(gtot_v[...], merged_v, r)
        r = r - habove
        wstar = jnp.left_shift(bsel, 24)

        cbase = lanes * PL

        @plsc.parallel_loop(0, NV, unroll=8, carry=(zvec, lanes))
        def comp0(i, c):
            cnt, jvec = c
            b = plsc.bitcast(row_v[pl.ds(i * L, L)], jnp.int32)
            hb = lax.shift_right_logical(b, 24)
            d = jnp.where(b < 0, hb, 127 - hb)
            m = d == bsel
            plsc.store_scatter(cand_v, [cbase + cnt], jvec, mask=m)
            return cnt + jnp.where(m, 1, 0), jvec + L
        cnt = comp0[0]

        for k in (1, 2, 3):
            shift = 24 - 8 * k
            trips = jnp.max(cnt)
            zero_hist()
            ghist_v[...] = zvec

            @plsc.parallel_loop(0, trips, unroll=2)
            def _(t, cnt=cnt, shift=shift):
                valid = t < cnt
                idxv = plsc.load_gather(cand_v, [cbase + t], mask=valid)
                wv = _key(plsc.bitcast(
                    plsc.load_gather(row_v, [idxv], mask=valid), jnp.int32))
                d = jnp.right_shift(wv, shift) & FF
                plsc.addupdate_scatter(hist_v, [d], ones, mask=valid)
                plsc.addupdate_scatter(
                    ghist_v, [jnp.right_shift(d, 4)], ones, mask=valid)

            bsel, habove = _locate(ghist_v[...], hist_v, r)
            r = r - habove
            wstar = wstar | jnp.left_shift(bsel, shift)

            def comp_k(t, cnt2, cnt=cnt, shift=shift, bsel=bsel):
                valid = t < cnt
                idxv = plsc.load_gather(cand_v, [cbase + t], mask=valid)
                wv = _key(plsc.bitcast(
                    plsc.load_gather(row_v, [idxv], mask=valid), jnp.int32))
                d = jnp.right_shift(wv, shift) & FF
                m = valid & (d == bsel)
                plsc.store_scatter(cand_v, [cbase + cnt2], idxv, mask=m)
                return cnt2 + jnp.where(m, 1, 0)
            cnt = lax.fori_loop(0, trips, comp_k, zvec)

        trips = jnp.max(cnt)

        def count_le(c2):
            def cbody(t, acc):
                valid = t < cnt
                idxv = plsc.load_gather(cand_v, [cbase + t], mask=valid)
                return acc + jnp.sum(jnp.where(valid & (idxv <= c2), 1, 0))
            return lax.fori_loop(0, trips, cbody, jnp.int32(0))

        def bsearch(i, c):
            c2 = c + jnp.left_shift(jnp.int32(1), 14 - i)
            return jnp.where(count_le(c2) < r, c2, c)
        cutoff = lax.fori_loop(0, 15, bsearch, jnp.int32(-1)) + 1

        tsigned = wstar ^ MININT
        pack = jnp.where(lanes == 2 * rr, tsigned, pack)
        pack = jnp.where(lanes == 2 * rr + 1, cutoff, pack)

    pack_v[...] = pack
    pltpu.sync_copy(pack_v, out_hbm.at[wid])


def _sc_select(x):
    mesh = plsc.VectorSubcoreMesh(core_axis_name="c", subcore_axis_name="s")
    return pl.kernel(
        _sc_body,
        out_type=jax.ShapeDtypeStruct((NW, L), jnp.int32),
        mesh=mesh,
        compiler_params=pltpu.CompilerParams(needs_layout_passes=False),
        scratch_types=[
            pltpu.VMEM((N,), jnp.float32),
            pltpu.VMEM((N,), jnp.float32),
            pltpu.VMEM((N + L,), jnp.int32),
            pltpu.VMEM((256 * L,), jnp.int32),
            pltpu.VMEM((256,), jnp.int32),
            pltpu.VMEM((256,), jnp.int32),
            pltpu.VMEM((L,), jnp.int32),
            pltpu.VMEM((L,), jnp.int32),
            pltpu.VMEM((L,), jnp.int32),
            pltpu.SemaphoreType.DMA,
            pltpu.SemaphoreType.DMA,
        ],
    )(x)


RB = 16


def _tc_body(x_ref, t_ref, c_ref, o_ref):
    xb = x_ref[...]
    b = lax.bitcast_convert_type(xb, jnp.int32)
    ws = _key(b) ^ MININT
    col = lax.broadcasted_iota(jnp.int32, xb.shape, 1)
    keep = (ws < t_ref[...]) | ((ws == t_ref[...]) & (col <= c_ref[...]))
    o_ref[...] = jnp.where(keep, jnp.maximum(xb, 0.0), 0.0)


def _tc_mask(x, t, c):
    return pl.pallas_call(
        _tc_body,
        grid=(B // RB,),
        in_specs=[
            pl.BlockSpec((RB, N), lambda i: (i, 0)),
            pl.BlockSpec((RB, 1), lambda i: (i, 0)),
            pl.BlockSpec((RB, 1), lambda i: (i, 0)),
        ],
        out_specs=pl.BlockSpec((RB, N), lambda i: (i, 0)),
        out_shape=jax.ShapeDtypeStruct((B, N), jnp.float32),
    )(x, t, c)


def kernel(x):
    packed = _sc_select(x)
    pairs = packed[:, : 2 * RPW].reshape(B, 2)
    return _tc_mask(x, pairs[:, 0:1], pairs[:, 1:2])


# speedup vs baseline: 1.1154x; 1.1154x over previous
import jax
import jax.numpy as jnp
from jax import lax
from jax.experimental import pallas as pl
from jax.experimental.pallas import tpu as pltpu
from jax.experimental.pallas import tpu_sc as plsc

K = 512
B, N = 128, 32768
NC, NS, L = 2, 16, 16
NW = NC * NS
RPW = B // NW
NV = N // L
PL = N // L
MASK7F = 0x7FFFFFFF
MININT = -2147483648
FF = 0xFF


def _key(b):
    m = jnp.right_shift(b, 31)
    return b ^ (~m & MASK7F)


def _locate(gt, hist_ref, r, L=16):
    cst = plsc.cumsum(gt)
    mlt = cst < r
    gs = plsc.all_reduce_population_count(mlt)[0]
    run = jnp.max(jnp.where(mlt, cst, 0))
    v = hist_ref[pl.ds(gs * L, L)]
    cs = plsc.cumsum(v) + run
    m2 = cs < r
    bw = plsc.all_reduce_population_count(m2)[0]
    habove = jnp.max(jnp.where(m2, cs, run))
    return gs * L + bw, habove


def _sc_body(x_hbm, out_hbm, rowa_v, rowb_v, cand_v, lh_v, merged_v, hist_v,
             gtot_v, ghist_v, sema, semb, semc, semd):
    wid = lax.axis_index("s") * NC + lax.axis_index("c")
    lanes = lax.iota(jnp.int32, L)
    ones = jnp.ones((L,), jnp.int32)
    zvec = jnp.zeros((L,), jnp.int32)

    rows = [rowa_v, rowb_v]
    sems = [sema, semb]
    osems = [semc, semd]
    copies = [None, None]
    ocopies = [None, None]
    copies[0] = pltpu.async_copy(x_hbm.at[wid * RPW], rowa_v, sema)

    def zero_hist():
        for g in range(256 // L):
            hist_v[pl.ds(g * L, L)] = zvec

    for rr in range(RPW):
        row_v = rows[rr % 2]
        if rr + 1 < RPW:
            if rr >= 1:
                ocopies[(rr + 1) % 2].wait()
            copies[(rr + 1) % 2] = pltpu.async_copy(
                x_hbm.at[wid * RPW + rr + 1], rows[(rr + 1) % 2],
                sems[(rr + 1) % 2])
        copies[rr % 2].wait()

        @plsc.parallel_loop(0, 256 * L // L, unroll=4)
        def _(g):
            lh_v[pl.ds(g * L, L)] = zvec

        lane_base = lanes * 256

        lb127 = lane_base + 127

        @plsc.parallel_loop(0, NV, unroll=8)
        def _(i):
            b = plsc.bitcast(row_v[pl.ds(i * L, L)], jnp.int32)
            hb = lax.shift_right_logical(b, 24)
            idx = jnp.where(b < 0, lane_base + hb, lb127 - hb)
            plsc.addupdate_scatter(lh_v, [idx], ones)

        r = jnp.int32(K)
        lane0 = lanes == 0

        @plsc.parallel_loop(0, 256 // L, unroll=2)
        def _(g):
            v = lh_v[pl.ds(g * L, L)]
            for l in range(1, L):
                v = v + lh_v[pl.ds(l * 256 + g * L, L)]
            merged_v[pl.ds(g * L, L)] = v
            tot = jnp.sum(v)
            plsc.store_scatter(gtot_v, [zvec + g], zvec + tot, mask=lane0)

        bsel, habove = _locate(gtot_v[...], merged_v, r)
        r = r - habove
        wstar = jnp.left_shift(bsel, 24)

        cbase = lanes * PL

        @plsc.parallel_loop(0, NV, unroll=8, carry=(zvec, lanes))
        def comp0(i, c):
            cnt, jvec = c
            b = plsc.bitcast(row_v[pl.ds(i * L, L)], jnp.int32)
            hb = lax.shift_right_logical(b, 24)
            d = jnp.where(b < 0, hb, 127 - hb)
            m = d == bsel
            plsc.store_scatter(cand_v, [cbase + cnt], jvec, mask=m)
            return cnt + jnp.where(m, 1, 0), jvec + L
        cnt = comp0[0]

        for k in (1, 2, 3):
            shift = 24 - 8 * k
            trips = jnp.max(cnt)
            zero_hist()
            ghist_v[...] = zvec

            @plsc.parallel_loop(0, trips, unroll=2)
            def _(t, cnt=cnt, shift=shift):
                valid = t < cnt
                idxv = plsc.load_gather(cand_v, [cbase + t], mask=valid)
                wv = _key(plsc.bitcast(
                    plsc.load_gather(row_v, [idxv], mask=valid), jnp.int32))
                d = jnp.right_shift(wv, shift) & FF
                plsc.addupdate_scatter(hist_v, [d], ones, mask=valid)
                plsc.addupdate_scatter(
                    ghist_v, [jnp.right_shift(d, 4)], ones, mask=valid)

            bsel, habove = _locate(ghist_v[...], hist_v, r)
            r = r - habove
            wstar = wstar | jnp.left_shift(bsel, shift)

            def comp_k(t, cnt2, cnt=cnt, shift=shift, bsel=bsel):
                valid = t < cnt
                idxv = plsc.load_gather(cand_v, [cbase + t], mask=valid)
                wv = _key(plsc.bitcast(
                    plsc.load_gather(row_v, [idxv], mask=valid), jnp.int32))
                d = jnp.right_shift(wv, shift) & FF
                m = valid & (d == bsel)
                plsc.store_scatter(cand_v, [cbase + cnt2], idxv, mask=m)
                return cnt2 + jnp.where(m, 1, 0)
            cnt = lax.fori_loop(0, trips, comp_k, zvec)

        trips = jnp.max(cnt)

        def count_le(c2):
            def cbody(t, acc):
                valid = t < cnt
                idxv = plsc.load_gather(cand_v, [cbase + t], mask=valid)
                return acc + jnp.sum(jnp.where(valid & (idxv <= c2), 1, 0))
            return lax.fori_loop(0, trips, cbody, jnp.int32(0))

        def bsearch(i, c):
            c2 = c + jnp.left_shift(jnp.int32(1), 14 - i)
            return jnp.where(count_le(c2) < r, c2, c)
        cutoff = lax.fori_loop(0, 15, bsearch, jnp.int32(-1)) + 1

        bstar = wstar ^ (~jnp.right_shift(wstar, 31) & MASK7F)
        tfv = plsc.bitcast(zvec + bstar, jnp.float32)
        relu_tf = jnp.maximum(tfv, 0.0)

        def eqc(t, off):
            valid = t < cnt
            idxv = plsc.load_gather(cand_v, [cbase + t], mask=valid)
            m = valid & (idxv <= cutoff)
            plsc.store_compressed(lh_v.at[pl.ds(off, L)], idxv, mask=m)
            return off + jnp.sum(jnp.where(m, 1, 0))
        noff = lax.fori_loop(0, trips, eqc, jnp.int32(0))

        @plsc.parallel_loop(0, NV, unroll=8)
        def _(i):
            xv = row_v[pl.ds(i * L, L)]
            row_v[pl.ds(i * L, L)] = jnp.where(
                xv > tfv, jnp.maximum(xv, 0.0), 0.0)

        def fix(t, carry):
            pos = t * L + lanes
            valid = pos < noff
            idxv = lh_v[pl.ds(t * L, L)]
            plsc.store_scatter(row_v, [idxv], relu_tf, mask=valid)
            return carry
        lax.fori_loop(0, (noff + L - 1) // L, fix, jnp.int32(0))

        ocopies[rr % 2] = pltpu.async_copy(
            row_v, out_hbm.at[wid * RPW + rr], osems[rr % 2])

    ocopies[0].wait()
    ocopies[1].wait()


def _sc_select(x):
    mesh = plsc.VectorSubcoreMesh(core_axis_name="c", subcore_axis_name="s")
    return pl.kernel(
        _sc_body,
        out_type=jax.ShapeDtypeStruct((B, N), jnp.float32),
        mesh=mesh,
        compiler_params=pltpu.CompilerParams(needs_layout_passes=False),
        scratch_types=[
            pltpu.VMEM((N,), jnp.float32),
            pltpu.VMEM((N,), jnp.float32),
            pltpu.VMEM((N + L,), jnp.int32),
            pltpu.VMEM((256 * L,), jnp.int32),
            pltpu.VMEM((256,), jnp.int32),
            pltpu.VMEM((256,), jnp.int32),
            pltpu.VMEM((L,), jnp.int32),
            pltpu.VMEM((L,), jnp.int32),
            pltpu.SemaphoreType.DMA,
            pltpu.SemaphoreType.DMA,
            pltpu.SemaphoreType.DMA,
            pltpu.SemaphoreType.DMA,
        ],
    )(x)


def kernel(x):
    return _sc_select(x)


# speedup vs baseline: 1.1377x; 1.0201x over previous
import jax
import jax.numpy as jnp
from jax import lax
from jax.experimental import pallas as pl
from jax.experimental.pallas import tpu as pltpu
from jax.experimental.pallas import tpu_sc as plsc

K = 512
B, N = 128, 32768
NC, NS, L = 2, 16, 16
NW = NC * NS
RPW = B // NW
NV = N // L
PL = N // L
MASK7F = 0x7FFFFFFF
MININT = -2147483648
FF = 0xFF


def _key(b):
    m = jnp.right_shift(b, 31)
    return b ^ (~m & MASK7F)


def _locate(gt, hist_ref, r, L=16):
    cst = plsc.cumsum(gt)
    mlt = cst < r
    gs = plsc.all_reduce_population_count(mlt)[0]
    run = jnp.max(jnp.where(mlt, cst, 0))
    v = hist_ref[pl.ds(gs * L, L)]
    cs = plsc.cumsum(v) + run
    m2 = cs < r
    bw = plsc.all_reduce_population_count(m2)[0]
    habove = jnp.max(jnp.where(m2, cs, run))
    return gs * L + bw, habove


def _sc_body(x_hbm, out_hbm, rowa_v, rowb_v, cand_v, lh_v, merged_v, hist_v,
             gtot_v, ghist_v, sema, semb, semc, semd):
    wid = lax.axis_index("s") * NC + lax.axis_index("c")
    lanes = lax.iota(jnp.int32, L)
    ones = jnp.ones((L,), jnp.int32)
    zvec = jnp.zeros((L,), jnp.int32)

    rows = [rowa_v, rowb_v]
    sems = [sema, semb]
    osems = [semc, semd]
    copies = [None, None]
    ocopies = [None, None]
    copies[0] = pltpu.async_copy(x_hbm.at[wid * RPW], rowa_v, sema)

    def zero_hist():
        for g in range(256 // L):
            hist_v[pl.ds(g * L, L)] = zvec

    for rr in range(RPW):
        row_v = rows[rr % 2]
        if rr + 1 < RPW:
            if rr >= 1:
                ocopies[(rr + 1) % 2].wait()
            copies[(rr + 1) % 2] = pltpu.async_copy(
                x_hbm.at[wid * RPW + rr + 1], rows[(rr + 1) % 2],
                sems[(rr + 1) % 2])
        copies[rr % 2].wait()

        @plsc.parallel_loop(0, 256 * L // L, unroll=4)
        def _(g):
            lh_v[pl.ds(g * L, L)] = zvec

        lane_base = lanes * 256

        lb127 = lane_base + 127

        @plsc.parallel_loop(0, NV, unroll=8)
        def _(i):
            b = plsc.bitcast(row_v[pl.ds(i * L, L)], jnp.int32)
            hb = lax.shift_right_logical(b, 24)
            idx = jnp.where(b < 0, lane_base + hb, lb127 - hb)
            plsc.addupdate_scatter(lh_v, [idx], ones)

        r = jnp.int32(K)
        lane0 = lanes == 0

        @plsc.parallel_loop(0, 256 // L, unroll=2)
        def _(g):
            v = lh_v[pl.ds(g * L, L)]
            for l in range(1, L):
                v = v + lh_v[pl.ds(l * 256 + g * L, L)]
            merged_v[pl.ds(g * L, L)] = v
            tot = jnp.sum(v)
            plsc.store_scatter(gtot_v, [zvec + g], zvec + tot, mask=lane0)

        bsel, habove = _locate(gtot_v[...], merged_v, r)
        r = r - habove
        wstar = jnp.left_shift(bsel, 24)

        cbase = lanes * PL

        @plsc.parallel_loop(0, NV, unroll=8, carry=(zvec, lanes))
        def comp0(i, c):
            cnt, jvec = c
            b = plsc.bitcast(row_v[pl.ds(i * L, L)], jnp.int32)
            hb = lax.shift_right_logical(b, 24)
            d = jnp.where(b < 0, hb, 127 - hb)
            m = d == bsel
            plsc.store_scatter(cand_v, [cbase + cnt], jvec, mask=m)
            return cnt + jnp.where(m, 1, 0), jvec + L
        cnt = comp0[0]

        for k in (1, 2, 3):
            shift = 24 - 8 * k
            trips = jnp.max(cnt)
            zero_hist()
            ghist_v[...] = zvec

            @plsc.parallel_loop(0, trips, unroll=2)
            def _(t, cnt=cnt, shift=shift):
                valid = t < cnt
                idxv = plsc.load_gather(cand_v, [cbase + t], mask=valid)
                wv = _key(plsc.bitcast(
                    plsc.load_gather(row_v, [idxv], mask=valid), jnp.int32))
                d = jnp.right_shift(wv, shift) & FF
                plsc.addupdate_scatter(hist_v, [d], ones, mask=valid)
                plsc.addupdate_scatter(
                    ghist_v, [jnp.right_shift(d, 4)], ones, mask=valid)

            bsel, habove = _locate(ghist_v[...], hist_v, r)
            r = r - habove
            wstar = wstar | jnp.left_shift(bsel, shift)

            def comp_k(t, cnt2, cnt=cnt, shift=shift, bsel=bsel):
                valid = t < cnt
                idxv = plsc.load_gather(cand_v, [cbase + t], mask=valid)
                wv = _key(plsc.bitcast(
                    plsc.load_gather(row_v, [idxv], mask=valid), jnp.int32))
                d = jnp.right_shift(wv, shift) & FF
                m = valid & (d == bsel)
                plsc.store_scatter(cand_v, [cbase + cnt2], idxv, mask=m)
                return cnt2 + jnp.where(m, 1, 0)
            cnt = lax.fori_loop(0, trips, comp_k, zvec)

        trips = jnp.max(cnt)

        def count_le(c2):
            def cbody(t, acc):
                valid = t < cnt
                idxv = plsc.load_gather(cand_v, [cbase + t], mask=valid)
                return acc + jnp.sum(jnp.where(valid & (idxv <= c2), 1, 0))
            return lax.fori_loop(0, trips, cbody, jnp.int32(0))

        def bsearch(i, c):
            c2 = c + jnp.left_shift(jnp.int32(1), 14 - i)
            return jnp.where(count_le(c2) < r, c2, c)

        def run_bsearch():
            return lax.fori_loop(0, 15, bsearch, jnp.int32(-1)) + 1
        cutoff = lax.cond(jnp.sum(cnt) == r, lambda: jnp.int32(N), run_bsearch)

        bstar = wstar ^ (~jnp.right_shift(wstar, 31) & MASK7F)
        tfv = plsc.bitcast(zvec + bstar, jnp.float32)
        relu_tf = jnp.maximum(tfv, 0.0)

        def eqc(t, off):
            valid = t < cnt
            idxv = plsc.load_gather(cand_v, [cbase + t], mask=valid)
            m = valid & (idxv <= cutoff)
            plsc.store_compressed(lh_v.at[pl.ds(off, L)], idxv, mask=m)
            return off + jnp.sum(jnp.where(m, 1, 0))
        noff = lax.fori_loop(0, trips, eqc, jnp.int32(0))

        @plsc.parallel_loop(0, NV, unroll=8)
        def _(i):
            xv = row_v[pl.ds(i * L, L)]
            row_v[pl.ds(i * L, L)] = jnp.where(
                xv > tfv, jnp.maximum(xv, 0.0), 0.0)

        def fix(t, carry):
            pos = t * L + lanes
            valid = pos < noff
            idxv = lh_v[pl.ds(t * L, L)]
            plsc.store_scatter(row_v, [idxv], relu_tf, mask=valid)
            return carry
        lax.fori_loop(0, (noff + L - 1) // L, fix, jnp.int32(0))

        ocopies[rr % 2] = pltpu.async_copy(
            row_v, out_hbm.at[wid * RPW + rr], osems[rr % 2])

    ocopies[0].wait()
    ocopies[1].wait()


def _sc_select(x):
    mesh = plsc.VectorSubcoreMesh(core_axis_name="c", subcore_axis_name="s")
    return pl.kernel(
        _sc_body,
        out_type=jax.ShapeDtypeStruct((B, N), jnp.float32),
        mesh=mesh,
        compiler_params=pltpu.CompilerParams(needs_layout_passes=False),
        scratch_types=[
            pltpu.VMEM((N,), jnp.float32),
            pltpu.VMEM((N,), jnp.float32),
            pltpu.VMEM((N + L,), jnp.int32),
            pltpu.VMEM((256 * L,), jnp.int32),
            pltpu.VMEM((256,), jnp.int32),
            pltpu.VMEM((256,), jnp.int32),
            pltpu.VMEM((L,), jnp.int32),
            pltpu.VMEM((L,), jnp.int32),
            pltpu.SemaphoreType.DMA,
            pltpu.SemaphoreType.DMA,
            pltpu.SemaphoreType.DMA,
            pltpu.SemaphoreType.DMA,
        ],
    )(x)


def kernel(x):
    return _sc_select(x)


# speedup vs baseline: 1.1426x; 1.0043x over previous
import jax
import jax.numpy as jnp
from jax import lax
from jax.experimental import pallas as pl
from jax.experimental.pallas import tpu as pltpu
from jax.experimental.pallas import tpu_sc as plsc

K = 512
B, N = 128, 32768
NC, NS, L = 2, 16, 16
NW = NC * NS
RPW = B // NW
NV = N // L
PL = N // L
MASK7F = 0x7FFFFFFF
MININT = -2147483648
FF = 0xFF


def _key(b):
    m = jnp.right_shift(b, 31)
    return b ^ (~m & MASK7F)


def _locate(gt, hist_ref, r, L=16):
    cst = plsc.cumsum(gt)
    mlt = cst < r
    gs = plsc.all_reduce_population_count(mlt)[0]
    run = jnp.max(jnp.where(mlt, cst, 0))
    v = hist_ref[pl.ds(gs * L, L)]
    cs = plsc.cumsum(v) + run
    m2 = cs < r
    bw = plsc.all_reduce_population_count(m2)[0]
    habove = jnp.max(jnp.where(m2, cs, run))
    return gs * L + bw, habove


def _sc_body(x_hbm, out_hbm, rowa_v, rowb_v, cand_v, lh_v, merged_v, hist_v,
             gtot_v, ghist_v, sema, semb, semc, semd):
    wid = lax.axis_index("s") * NC + lax.axis_index("c")
    lanes = lax.iota(jnp.int32, L)
    ones = jnp.ones((L,), jnp.int32)
    zvec = jnp.zeros((L,), jnp.int32)

    rows = [rowa_v, rowb_v]
    sems = [sema, semb]
    osems = [semc, semd]
    copies = [None, None]
    ocopies = [None, None]
    copies[0] = pltpu.async_copy(x_hbm.at[wid * RPW], rowa_v, sema)

    def zero_hist():
        for g in range(256 // L):
            hist_v[pl.ds(g * L, L)] = zvec

    for rr in range(RPW):
        row_v = rows[rr % 2]
        if rr + 1 < RPW:
            if rr >= 1:
                ocopies[(rr + 1) % 2].wait()
            copies[(rr + 1) % 2] = pltpu.async_copy(
                x_hbm.at[wid * RPW + rr + 1], rows[(rr + 1) % 2],
                sems[(rr + 1) % 2])
        copies[rr % 2].wait()

        @plsc.parallel_loop(0, 256 * L // L, unroll=4)
        def _(g):
            lh_v[pl.ds(g * L, L)] = zvec

        lane_base = lanes * 256

        lb127 = lane_base + 127

        @plsc.parallel_loop(0, NV, unroll=16)
        def _(i):
            b = plsc.bitcast(row_v[pl.ds(i * L, L)], jnp.int32)
            hb = lax.shift_right_logical(b, 24)
            idx = jnp.where(b < 0, lane_base + hb, lb127 - hb)
            plsc.addupdate_scatter(lh_v, [idx], ones)

        r = jnp.int32(K)
        lane0 = lanes == 0

        @plsc.parallel_loop(0, 256 // L, unroll=2)
        def _(g):
            v = lh_v[pl.ds(g * L, L)]
            for l in range(1, L):
                v = v + lh_v[pl.ds(l * 256 + g * L, L)]
            merged_v[pl.ds(g * L, L)] = v
            tot = jnp.sum(v)
            plsc.store_scatter(gtot_v, [zvec + g], zvec + tot, mask=lane0)

        bsel, habove = _locate(gtot_v[...], merged_v, r)
        r = r - habove
        wstar = jnp.left_shift(bsel, 24)

        cbase = lanes * PL

        @plsc.parallel_loop(0, NV, unroll=8, carry=(zvec, lanes))
        def comp0(i, c):
            cnt, jvec = c
            b = plsc.bitcast(row_v[pl.ds(i * L, L)], jnp.int32)
            hb = lax.shift_right_logical(b, 24)
            d = jnp.where(b < 0, hb, 127 - hb)
            m = d == bsel
            plsc.store_scatter(cand_v, [cbase + cnt], jvec, mask=m)
            return cnt + jnp.where(m, 1, 0), jvec + L
        cnt = comp0[0]

        for k in (1, 2, 3):
            shift = 24 - 8 * k
            trips = jnp.max(cnt)
            zero_hist()
            ghist_v[...] = zvec

            @plsc.parallel_loop(0, trips, unroll=2)
            def _(t, cnt=cnt, shift=shift):
                valid = t < cnt
                idxv = plsc.load_gather(cand_v, [cbase + t], mask=valid)
                wv = _key(plsc.bitcast(
                    plsc.load_gather(row_v, [idxv], mask=valid), jnp.int32))
                d = jnp.right_shift(wv, shift) & FF
                plsc.addupdate_scatter(hist_v, [d], ones, mask=valid)
                plsc.addupdate_scatter(
                    ghist_v, [jnp.right_shift(d, 4)], ones, mask=valid)

            bsel, habove = _locate(ghist_v[...], hist_v, r)
            r = r - habove
            wstar = wstar | jnp.left_shift(bsel, shift)

            def comp_k(t, cnt2, cnt=cnt, shift=shift, bsel=bsel):
                valid = t < cnt
                idxv = plsc.load_gather(cand_v, [cbase + t], mask=valid)
                wv = _key(plsc.bitcast(
                    plsc.load_gather(row_v, [idxv], mask=valid), jnp.int32))
                d = jnp.right_shift(wv, shift) & FF
                m = valid & (d == bsel)
                plsc.store_scatter(cand_v, [cbase + cnt2], idxv, mask=m)
                return cnt2 + jnp.where(m, 1, 0)
            cnt = lax.fori_loop(0, trips, comp_k, zvec)

        trips = jnp.max(cnt)

        def count_le(c2):
            def cbody(t, acc):
                valid = t < cnt
                idxv = plsc.load_gather(cand_v, [cbase + t], mask=valid)
                return acc + jnp.sum(jnp.where(valid & (idxv <= c2), 1, 0))
            return lax.fori_loop(0, trips, cbody, jnp.int32(0))

        def bsearch(i, c):
            c2 = c + jnp.left_shift(jnp.int32(1), 14 - i)
            return jnp.where(count_le(c2) < r, c2, c)

        def run_bsearch():
            return lax.fori_loop(0, 15, bsearch, jnp.int32(-1)) + 1
        cutoff = lax.cond(jnp.sum(cnt) == r, lambda: jnp.int32(N), run_bsearch)

        bstar = wstar ^ (~jnp.right_shift(wstar, 31) & MASK7F)
        tfv = plsc.bitcast(zvec + bstar, jnp.float32)
        relu_tf = jnp.maximum(tfv, 0.0)

        def eqc(t, off):
            valid = t < cnt
            idxv = plsc.load_gather(cand_v, [cbase + t], mask=valid)
            m = valid & (idxv <= cutoff)
            plsc.store_compressed(lh_v.at[pl.ds(off, L)], idxv, mask=m)
            return off + jnp.sum(jnp.where(m, 1, 0))
        noff = lax.fori_loop(0, trips, eqc, jnp.int32(0))

        @plsc.parallel_loop(0, NV, unroll=16)
        def _(i):
            xv = row_v[pl.ds(i * L, L)]
            row_v[pl.ds(i * L, L)] = jnp.where(
                xv > tfv, jnp.maximum(xv, 0.0), 0.0)

        def fix(t, carry):
            pos = t * L + lanes
            valid = pos < noff
            idxv = lh_v[pl.ds(t * L, L)]
            plsc.store_scatter(row_v, [idxv], relu_tf, mask=valid)
            return carry
        lax.fori_loop(0, (noff + L - 1) // L, fix, jnp.int32(0))

        ocopies[rr % 2] = pltpu.async_copy(
            row_v, out_hbm.at[wid * RPW + rr], osems[rr % 2])

    ocopies[0].wait()
    ocopies[1].wait()


def _sc_select(x):
    mesh = plsc.VectorSubcoreMesh(core_axis_name="c", subcore_axis_name="s")
    return pl.kernel(
        _sc_body,
        out_type=jax.ShapeDtypeStruct((B, N), jnp.float32),
        mesh=mesh,
        compiler_params=pltpu.CompilerParams(needs_layout_passes=False),
        scratch_types=[
            pltpu.VMEM((N,), jnp.float32),
            pltpu.VMEM((N,), jnp.float32),
            pltpu.VMEM((N + L,), jnp.int32),
            pltpu.VMEM((256 * L,), jnp.int32),
            pltpu.VMEM((256,), jnp.int32),
            pltpu.VMEM((256,), jnp.int32),
            pltpu.VMEM((L,), jnp.int32),
            pltpu.VMEM((L,), jnp.int32),
            pltpu.SemaphoreType.DMA,
            pltpu.SemaphoreType.DMA,
            pltpu.SemaphoreType.DMA,
            pltpu.SemaphoreType.DMA,
        ],
    )(x)


def kernel(x):
    return _sc_select(x)
